# Initial kernel scaffold; baseline (speedup 1.0000x reference)
import jax
import jax.numpy as jnp
from jax.experimental import pallas as pl


def kernel(D, logm_planets, logG, W1, b1, W2, b2, W3, b3, W4, b4):
    raise NotImplementedError("write your pallas kernel here")


# speedup vs baseline: 4.5846x; 4.5846x over previous
import numpy as np
import jax
import jax.numpy as jnp
from jax.experimental import pallas as pl

_NP = 10
_NE = _NP * (_NP - 1) // 2
_sl, _rl = [], []
for _i in range(_NP):
    for _j in range(_i + 1, _NP):
        _sl.append(_i)
        _rl.append(_j)
_SEND = np.array(_sl, dtype=np.int32)
_RECV = np.array(_rl, dtype=np.int32)
_LOGM0 = float(np.log10(5.522376708530351))

_TB = 128

_MPAT = np.zeros((_NE * 3, _NP * 3), dtype=np.float32)
for _e in range(_NE):
    for _c in range(3):
        _MPAT[_e * 3 + _c, _RECV[_e] * 3 + _c] += 1.0
        _MPAT[_e * 3 + _c, _SEND[_e] * 3 + _c] -= 1.0


def _mlp_kernel(d_ref, w1_ref, b1p_ref, w2_ref, b2_ref, w3_ref, b3_ref,
                w4_ref, b4_ref, fc_ref):
    d = d_ref[...]
    x = d[:, 0:1]
    y = d[:, 1:2]
    z = d[:, 2:3]
    r = jnp.sqrt(x * x + y * y + z * z)
    w = jnp.clip(z / (r + 1e-12), -1.0, 1.0)
    theta = jnp.arctan2(jnp.sqrt(jnp.maximum(1.0 - w * w, 0.0)), w)
    phi = jnp.arctan2(y, x)
    sph = jnp.concatenate([r, theta, phi], axis=1)
    h = jnp.dot(sph, w1_ref[...], preferred_element_type=jnp.float32)
    h = jnp.maximum(h + b1p_ref[...], 0.0)
    h = jnp.dot(h, w2_ref[...], preferred_element_type=jnp.float32)
    h = jnp.maximum(h + b2_ref[...], 0.0)
    h = jnp.dot(h, w3_ref[...], preferred_element_type=jnp.float32)
    h = jnp.maximum(h + b3_ref[...], 0.0)
    oe = jnp.dot(h, w4_ref[...], preferred_element_type=jnp.float32) + b4_ref[...]
    rr = oe[:, 0:1]
    th = oe[:, 1:2]
    ph = oe[:, 2:3]
    st = jnp.sin(th)
    fx = rr * st * jnp.cos(ph)
    fy = rr * st * jnp.sin(ph)
    fz = rr * jnp.cos(th)
    fc_ref[...] = jnp.concatenate([fx, fy, fz], axis=1)


def _agg_kernel(g_ref, m_ref, o_ref):
    o_ref[...] = jnp.dot(g_ref[...], m_ref[...],
                         preferred_element_type=jnp.float32)


def kernel(D, logm_planets, logG, W1, b1, W2, b2, W3, b3, W4, b4):
    E = D.shape[0]
    nt = E // _NE
    rows = _TB * _NE

    lm = jnp.concatenate(
        [jnp.full((1,), _LOGM0, dtype=jnp.float32), logm_planets], axis=0)
    B1 = (lm[_RECV][:, None] * W1[3:4, :] + lm[_SEND][:, None] * W1[4:5, :]
          + logG[0] * W1[5:6, :] + b1[None, :])
    B1t = jnp.tile(B1, (_TB, 1))

    inv_mass = jnp.power(10.0, -lm)
    M2 = jnp.asarray(_MPAT) * jnp.repeat(inv_mass, 3)[None, :]

    fc = pl.pallas_call(
        _mlp_kernel,
        grid=(nt // _TB,),
        in_specs=[
            pl.BlockSpec((rows, 3), lambda i: (i, 0)),
            pl.BlockSpec((3, 128), lambda i: (0, 0)),
            pl.BlockSpec((rows, 128), lambda i: (0, 0)),
            pl.BlockSpec((128, 128), lambda i: (0, 0)),
            pl.BlockSpec((1, 128), lambda i: (0, 0)),
            pl.BlockSpec((128, 128), lambda i: (0, 0)),
            pl.BlockSpec((1, 128), lambda i: (0, 0)),
            pl.BlockSpec((128, 3), lambda i: (0, 0)),
            pl.BlockSpec((1, 3), lambda i: (0, 0)),
        ],
        out_specs=pl.BlockSpec((rows, 3), lambda i: (i, 0)),
        out_shape=jax.ShapeDtypeStruct((E, 3), jnp.float32),
    )(D, W1[:3, :], B1t, W2, b2[None, :], W3, b3[None, :], W4, b4[None, :])

    G = fc.reshape(nt, _NE * 3)

    out = pl.pallas_call(
        _agg_kernel,
        grid=(8,),
        in_specs=[
            pl.BlockSpec((nt // 8, _NE * 3), lambda i: (i, 0)),
            pl.BlockSpec((_NE * 3, _NP * 3), lambda i: (0, 0)),
        ],
        out_specs=pl.BlockSpec((nt // 8, _NP * 3), lambda i: (i, 0)),
        out_shape=jax.ShapeDtypeStruct((nt, _NP * 3), jnp.float32),
    )(G, M2)

    return out.reshape(nt * _NP, 3)


# speedup vs baseline: 41.5676x; 9.0668x over previous
import numpy as np
import jax
import jax.numpy as jnp
from jax.experimental import pallas as pl

_NP = 10
_NE = _NP * (_NP - 1) // 2
_sl, _rl = [], []
for _i in range(_NP):
    for _j in range(_i + 1, _NP):
        _sl.append(_i)
        _rl.append(_j)
_SEND = np.array(_sl, dtype=np.int32)
_RECV = np.array(_rl, dtype=np.int32)
_LOGM0 = float(np.log10(5.522376708530351))

_TB = 128

_SPAT = np.zeros((_NE, _NP), dtype=np.float32)
for _e in range(_NE):
    _SPAT[_e, _RECV[_e]] += 1.0
    _SPAT[_e, _SEND[_e]] -= 1.0


def _mlp_kernel(d_ref, w1_ref, b1p_ref, w2_ref, b2_ref, w3_ref, b3_ref,
                w4_ref, b4_ref, fc_ref):
    d = d_ref[...]
    x = d[0:1, :]
    y = d[1:2, :]
    z = d[2:3, :]
    r = jnp.sqrt(x * x + y * y + z * z)
    w = jnp.clip(z / (r + 1e-12), -1.0, 1.0)
    theta = jnp.arctan2(jnp.sqrt(jnp.maximum(1.0 - w * w, 0.0)), w)
    phi = jnp.arctan2(y, x)
    sph = jnp.concatenate([r, theta, phi], axis=0)
    h = jnp.dot(w1_ref[...], sph, preferred_element_type=jnp.float32)
    h = jnp.maximum(h + b1p_ref[...], 0.0)
    h = jnp.dot(w2_ref[...], h, preferred_element_type=jnp.float32)
    h = jnp.maximum(h + b2_ref[...], 0.0)
    h = jnp.dot(w3_ref[...], h, preferred_element_type=jnp.float32)
    h = jnp.maximum(h + b3_ref[...], 0.0)
    oe = jnp.dot(w4_ref[...], h, preferred_element_type=jnp.float32)
    oe = oe + b4_ref[...]
    rr = oe[0:1, :]
    th = oe[1:2, :]
    ph = oe[2:3, :]
    st = jnp.sin(th)
    fx = rr * st * jnp.cos(ph)
    fy = rr * st * jnp.sin(ph)
    fz = rr * jnp.cos(th)
    fc_ref[...] = jnp.concatenate([fx, fy, fz], axis=0)


def _agg_kernel(g_ref, m_ref, o_ref):
    for c in range(3):
        o_ref[c] = jnp.dot(g_ref[c], m_ref[...],
                           preferred_element_type=jnp.float32)


def kernel(D, logm_planets, logG, W1, b1, W2, b2, W3, b3, W4, b4):
    E = D.shape[0]
    nt = E // _NE
    lanes = _TB * _NE

    lm = jnp.concatenate(
        [jnp.full((1,), _LOGM0, dtype=jnp.float32), logm_planets], axis=0)
    B1 = (lm[_RECV][:, None] * W1[3:4, :] + lm[_SEND][:, None] * W1[4:5, :]
          + logG[0] * W1[5:6, :] + b1[None, :])
    B1t = jnp.tile(B1.T, (1, _TB))

    inv_mass = jnp.power(10.0, -lm)
    M2 = jnp.asarray(_SPAT) * inv_mass[None, :]

    DT = D.T

    fcT = pl.pallas_call(
        _mlp_kernel,
        grid=(nt // _TB,),
        in_specs=[
            pl.BlockSpec((3, lanes), lambda i: (0, i)),
            pl.BlockSpec((128, 3), lambda i: (0, 0)),
            pl.BlockSpec((128, lanes), lambda i: (0, 0)),
            pl.BlockSpec((128, 128), lambda i: (0, 0)),
            pl.BlockSpec((128, 1), lambda i: (0, 0)),
            pl.BlockSpec((128, 128), lambda i: (0, 0)),
            pl.BlockSpec((128, 1), lambda i: (0, 0)),
            pl.BlockSpec((3, 128), lambda i: (0, 0)),
            pl.BlockSpec((3, 1), lambda i: (0, 0)),
        ],
        out_specs=pl.BlockSpec((3, lanes), lambda i: (0, i)),
        out_shape=jax.ShapeDtypeStruct((3, E), jnp.float32),
    )(DT, W1[:3, :].T, B1t, W2.T, b2[:, None], W3.T, b3[:, None],
      W4.T, b4[:, None])

    G = fcT.reshape(3, nt, _NE)

    out = pl.pallas_call(
        _agg_kernel,
        grid=(8,),
        in_specs=[
            pl.BlockSpec((3, nt // 8, _NE), lambda i: (0, i, 0)),
            pl.BlockSpec((_NE, _NP), lambda i: (0, 0)),
        ],
        out_specs=pl.BlockSpec((3, nt // 8, _NP), lambda i: (0, i, 0)),
        out_shape=jax.ShapeDtypeStruct((3, nt, _NP), jnp.float32),
    )(G, M2)

    return out.transpose(1, 2, 0).reshape(nt * _NP, 3)
